# Initial kernel scaffold; baseline (speedup 1.0000x reference)
import jax
import jax.numpy as jnp
from jax.experimental import pallas as pl


def kernel(nodes_u, nodes_v, hist_u_v, hist_u_r, hist_v_u, hist_v_r, social_adj, params):
    raise NotImplementedError("write your pallas kernel here")


# speedup vs baseline: 2.5048x; 2.5048x over previous
import functools

import jax
import jax.numpy as jnp
from jax import lax
from jax.experimental import pallas as pl
from jax.experimental.pallas import tpu as pltpu
from jax.experimental.pallas import tpu_sc as plsc

B = 4096
Hn = 50
Dm = 64
NR = 5

NW = 32
BPW = B // NW
RPW = BPW * Hn
CHK = 128
NCHK = RPW // CHK
NBUF = 4

BB = 64
NB = B // BB


def _sc_gather(nodes_u, nodes_v, hu_ids, soc_ids, hv_ids, u2e, v2e):
    mesh = plsc.VectorSubcoreMesh(core_axis_name="c", subcore_axis_name="s")
    out_type = (
        jax.ShapeDtypeStruct((B * Hn, Dm), jnp.float32),
        jax.ShapeDtypeStruct((B * Hn, Dm), jnp.float32),
        jax.ShapeDtypeStruct((B * Hn, Dm), jnp.float32),
        jax.ShapeDtypeStruct((B, Dm), jnp.float32),
        jax.ShapeDtypeStruct((B, Dm), jnp.float32),
    )
    scratch = [
        pltpu.VMEM((BPW,), jnp.int32),
        pltpu.VMEM((BPW,), jnp.int32),
        pltpu.VMEM((RPW,), jnp.int32),
        pltpu.VMEM((BPW, Dm), jnp.float32),
        pltpu.VMEM((NBUF, CHK, Dm), jnp.float32),
        pltpu.SemaphoreType.DMA,
        pltpu.SemaphoreType.DMA,
    ]

    @functools.partial(
        pl.kernel, mesh=mesh, out_type=out_type, scratch_types=scratch,
        compiler_params=pltpu.CompilerParams(use_tc_tiling_on_sc=False))
    def k(nodes_u_h, nodes_v_h, hu_h, so_h, hv_h, u2e_h, v2e_h,
          ehu_o, eso_o, ehv_o, urep_o, vrep_o,
          nu_v, nv_v, ids_v, rep_v, emb_v, gsem, wsem):
        wid = lax.axis_index("s") * 2 + lax.axis_index("c")
        base = wid * BPW
        pltpu.sync_copy(nodes_u_h.at[pl.ds(base, BPW)], nu_v)
        pltpu.sync_copy(nodes_v_h.at[pl.ds(base, BPW)], nv_v)
        pltpu.async_copy(u2e_h.at[nu_v], rep_v, gsem).wait()
        pltpu.sync_copy(rep_v, urep_o.at[pl.ds(base, BPW)])
        pltpu.async_copy(v2e_h.at[nv_v], rep_v, gsem).wait()
        pltpu.sync_copy(rep_v, vrep_o.at[pl.ds(base, BPW)])

        def gather_rows(ids_hbm, emb_tbl, out_ref):
            pltpu.sync_copy(ids_hbm.at[pl.ds(base * Hn, RPW)], ids_v)
            obase = base * Hn

            def gstart(c, b):
                pltpu.async_copy(
                    emb_tbl.at[ids_v.at[pl.ds(c * CHK, CHK)]],
                    emb_v.at[b], gsem)

            def gwait(b):
                pltpu.make_async_copy(
                    emb_tbl.at[pl.ds(0, CHK)], emb_v.at[b], gsem).wait()

            def wstart(c, b):
                pltpu.async_copy(
                    emb_v.at[b], out_ref.at[pl.ds(obase + c * CHK, CHK)],
                    wsem)

            def wwait(b):
                pltpu.make_async_copy(
                    emb_v.at[b], out_ref.at[pl.ds(obase, CHK)], wsem).wait()

            def body(c, carry):
                b = lax.rem(c, NBUF)

                @pl.when(c >= NBUF)
                def _():
                    wwait(b)

                gstart(c, b)

                @pl.when(c >= 1)
                def _():
                    bp = lax.rem(c - 1, NBUF)
                    gwait(bp)
                    wstart(c - 1, bp)
                return carry

            lax.fori_loop(0, NCHK, body, 0)
            bl = (NCHK - 1) % NBUF
            gwait(bl)
            wstart(NCHK - 1, bl)
            for _ in range(min(NBUF, NCHK)):
                wwait(0)

        gather_rows(hu_h, v2e_h, ehu_o)
        gather_rows(so_h, u2e_h, eso_o)
        gather_rows(hv_h, u2e_h, ehv_o)

    return k(nodes_u, nodes_v, hu_ids, soc_ids, hv_ids, u2e, v2e)


def _dot(x, w):
    return jax.lax.dot_general(x, w, (((1,), (0,)), ((), ())),
                               preferred_element_type=jnp.float32)


def _tc_enc_body(ehu_r, eso_r, ehv_r, hur_r, hvr_r, urep_r, vrep_r, r2e_r,
                 ua_w1_r, ua_b1_r, ua_w2_r, ua_b2_r,
                 ua_a1_r, ua_ab1_r, ua_a2_r, ua_ab2_r, ua_a3_r, ua_ab3_r,
                 ue_w_r, ue_b_r,
                 sa_a1_r, sa_ab1_r, sa_a2_r, sa_ab2_r, sa_a3_r, sa_ab3_r,
                 se_w_r, se_b_r,
                 va_w1_r, va_b1_r, va_w2_r, va_b2_r,
                 va_a1_r, va_ab1_r, va_a2_r, va_ab2_r, va_a3_r, va_ab3_r,
                 ve_w_r, ve_b_r,
                 eu_o, ev_o):
    r2e = r2e_r[...]

    def rating_emb(ids):
        acc = jnp.zeros((BB, Hn, Dm), jnp.float32)
        for k in range(NR):
            m = (ids == k).astype(jnp.float32)[:, :, None]
            acc = acc + m * r2e[k][None, None, :]
        return acc

    def attention(o_flat, rep, a1, ab1, a2, ab2, a3, ab3):
        repb = jnp.broadcast_to(rep[:, None, :], (BB, Hn, Dm))
        repb = repb.reshape(BB * Hn, Dm)
        a = jnp.concatenate([o_flat, repb], axis=-1)
        a = jnp.maximum(_dot(a, a1[...]) + ab1[...], 0.0)
        a = jnp.maximum(_dot(a, a2[...]) + ab2[...], 0.0)
        lg = jnp.sum(a * a3[...], axis=-1) + ab3[0, 0]
        lg = lg.reshape(BB, Hn)
        m = jnp.max(lg, axis=1, keepdims=True)
        e = jnp.exp(lg - m)
        return e / jnp.sum(e, axis=1, keepdims=True)

    def uv_encode(e_uv, ids, rep, w1, b1, w2, b2,
                  a1, ab1, a2, ab2, a3, ab3, ew, eb):
        e_r = rating_emb(ids)
        x = jnp.concatenate([e_uv, e_r], axis=-1).reshape(BB * Hn, 2 * Dm)
        x = jnp.maximum(_dot(x, w1[...]) + b1[...], 0.0)
        o = jnp.maximum(_dot(x, w2[...]) + b2[...], 0.0)
        att = attention(o, rep, a1, ab1, a2, ab2, a3, ab3)
        o3 = o.reshape(BB, Hn, Dm)
        agg = jnp.sum(att[:, :, None] * o3, axis=1)
        cat = jnp.concatenate([rep, agg], axis=1)
        return jnp.maximum(_dot(cat, ew[...]) + eb[...], 0.0)

    urep = urep_r[...]
    vrep = vrep_r[...]
    u_hist = uv_encode(ehu_r[...], hur_r[...], urep,
                       ua_w1_r, ua_b1_r, ua_w2_r, ua_b2_r,
                       ua_a1_r, ua_ab1_r, ua_a2_r, ua_ab2_r, ua_a3_r, ua_ab3_r,
                       ue_w_r, ue_b_r)
    neigh = eso_r[...]
    aw = attention(neigh.reshape(BB * Hn, Dm), urep,
                   sa_a1_r, sa_ab1_r, sa_a2_r, sa_ab2_r, sa_a3_r, sa_ab3_r)
    soc_agg = jnp.sum(aw[:, :, None] * neigh, axis=1)
    cat = jnp.concatenate([u_hist, soc_agg], axis=1)
    eu_o[...] = jnp.maximum(_dot(cat, se_w_r[...]) + se_b_r[...], 0.0)

    ev_o[...] = uv_encode(ehv_r[...], hvr_r[...], vrep,
                          va_w1_r, va_b1_r, va_w2_r, va_b2_r,
                          va_a1_r, va_ab1_r, va_a2_r, va_ab2_r,
                          va_a3_r, va_ab3_r,
                          ve_w_r, ve_b_r)


def _tc_encoders(ehu, eso, ehv, hur, hvr, urep, vrep, r2e, wts):
    def b3(shape):
        return pl.BlockSpec(shape, lambda i: (i, 0, 0))

    def b2(shape):
        return pl.BlockSpec(shape, lambda i: (i, 0))

    def full2(a):
        return pl.BlockSpec(a.shape, lambda i: (0, 0))

    in_specs = ([b3((BB, Hn, Dm))] * 3 + [b2((BB, Hn))] * 2 +
                [b2((BB, Dm))] * 2 + [full2(r2e)] +
                [full2(w) for w in wts])
    out_specs = (b2((BB, Dm)), b2((BB, Dm)))
    out_shape = (jax.ShapeDtypeStruct((B, Dm), jnp.float32),
                 jax.ShapeDtypeStruct((B, Dm), jnp.float32))
    return pl.pallas_call(
        _tc_enc_body,
        grid=(NB,),
        in_specs=in_specs,
        out_specs=out_specs,
        out_shape=out_shape,
    )(ehu, eso, ehv, hur, hvr, urep, vrep, r2e, *wts)


def _tc_head_body(eu_r, ev_r,
                  ur1_w_r, ur1_b_r, ur2_w_r, ur2_b_r,
                  vr1_w_r, vr1_b_r, vr2_w_r, vr2_b_r,
                  uv1_w_r, uv1_b_r, uv2_w_r, uv2_b_r, uv3_w_r, uv3_b_r,
                  bn1_g_r, bn1_b_r, bn2_g_r, bn2_b_r,
                  bn3_g_r, bn3_b_r, bn4_g_r, bn4_b_r,
                  out_o):
    def bn(x, g, b):
        m = jnp.mean(x, axis=0, keepdims=True)
        v = jnp.mean((x - m) * (x - m), axis=0, keepdims=True)
        return g[...] * (x - m) * jax.lax.rsqrt(v + 1e-5) + b[...]

    x_u = jnp.maximum(bn(_dot(eu_r[...], ur1_w_r[...]) + ur1_b_r[...],
                         bn1_g_r, bn1_b_r), 0.0)
    x_u = _dot(x_u, ur2_w_r[...]) + ur2_b_r[...]
    x_v = jnp.maximum(bn(_dot(ev_r[...], vr1_w_r[...]) + vr1_b_r[...],
                         bn2_g_r, bn2_b_r), 0.0)
    x_v = _dot(x_v, vr2_w_r[...]) + vr2_b_r[...]
    x = jnp.concatenate([x_u, x_v], axis=1)
    x = jnp.maximum(bn(_dot(x, uv1_w_r[...]) + uv1_b_r[...],
                       bn3_g_r, bn3_b_r), 0.0)
    x = jnp.maximum(bn(_dot(x, uv2_w_r[...]) + uv2_b_r[...],
                       bn4_g_r, bn4_b_r), 0.0)
    out_o[...] = jnp.sum(x * uv3_w_r[...], axis=-1, keepdims=True) \
        + uv3_b_r[0, 0]


def _tc_head(eu, ev, wts):
    in_specs = [pl.BlockSpec(a.shape, lambda: tuple(0 for _ in a.shape))
                for a in (eu, ev, *wts)]
    return pl.pallas_call(
        _tc_head_body,
        in_specs=in_specs,
        out_specs=pl.BlockSpec((B, 1), lambda: (0, 0)),
        out_shape=jax.ShapeDtypeStruct((B, 1), jnp.float32),
    )(eu, ev, *wts)


def kernel(nodes_u, nodes_v, hist_u_v, hist_u_r, hist_v_u, hist_v_r,
           social_adj, params):
    p = params
    hu_ids = jnp.take(hist_u_v, nodes_u, axis=0).reshape(-1)
    soc_ids = jnp.take(social_adj, nodes_u, axis=0).reshape(-1)
    hv_ids = jnp.take(hist_v_u, nodes_v, axis=0).reshape(-1)
    hur = jnp.take(hist_u_r, nodes_u, axis=0)
    hvr = jnp.take(hist_v_r, nodes_v, axis=0)
    ehu, eso, ehv, urep, vrep = _sc_gather(
        nodes_u, nodes_v, hu_ids, soc_ids, hv_ids, p['u2e'], p['v2e'])
    ehu = ehu.reshape(B, Hn, Dm)
    eso = eso.reshape(B, Hn, Dm)
    ehv = ehv.reshape(B, Hn, Dm)

    def row(v):
        return v.reshape(1, -1)

    ua, sa, va = p['ua'], p['sa'], p['va']
    enc_wts = [
        ua['w_r1_w'], row(ua['w_r1_b']), ua['w_r2_w'], row(ua['w_r2_b']),
        ua['att1_w'], row(ua['att1_b']), ua['att2_w'], row(ua['att2_b']),
        ua['att3_w'].T, row(ua['att3_b']),
        p['ue_w'], row(p['ue_b']),
        sa['att1_w'], row(sa['att1_b']), sa['att2_w'], row(sa['att2_b']),
        sa['att3_w'].T, row(sa['att3_b']),
        p['se_w'], row(p['se_b']),
        va['w_r1_w'], row(va['w_r1_b']), va['w_r2_w'], row(va['w_r2_b']),
        va['att1_w'], row(va['att1_b']), va['att2_w'], row(va['att2_b']),
        va['att3_w'].T, row(va['att3_b']),
        p['ve_w'], row(p['ve_b']),
    ]
    eu, ev = _tc_encoders(ehu, eso, ehv, hur, hvr, urep, vrep,
                          p['r2e'], enc_wts)

    head_wts = [
        p['w_ur1_w'], row(p['w_ur1_b']), p['w_ur2_w'], row(p['w_ur2_b']),
        p['w_vr1_w'], row(p['w_vr1_b']), p['w_vr2_w'], row(p['w_vr2_b']),
        p['w_uv1_w'], row(p['w_uv1_b']), p['w_uv2_w'], row(p['w_uv2_b']),
        p['w_uv3_w'].T, row(p['w_uv3_b']),
        row(p['bn1_g']), row(p['bn1_b']), row(p['bn2_g']), row(p['bn2_b']),
        row(p['bn3_g']), row(p['bn3_b']), row(p['bn4_g']), row(p['bn4_b']),
    ]
    out = _tc_head(eu, ev, head_wts)
    return out.reshape(B)


# speedup vs baseline: 8.0945x; 3.2316x over previous
import functools

import jax
import jax.numpy as jnp
from jax import lax
from jax.experimental import pallas as pl
from jax.experimental.pallas import tpu as pltpu
from jax.experimental.pallas import tpu_sc as plsc

B = 4096
Bp = B // 2
Hn = 50
Dm = 64
Dp = 2 * Dm
NR = 5

NW = 32
BPW = B // NW
RPW = BPW * Hn
CHK = 128
NCHK = RPW // CHK
NBUF = 4

BBp = 128
NB = Bp // BBp


def _sc_gather(nodes_u, nodes_v, hu_ids, soc_ids, hv_ids, u2e, v2e):
    mesh = plsc.VectorSubcoreMesh(core_axis_name="c", subcore_axis_name="s")
    out_type = (
        jax.ShapeDtypeStruct((Hn * Bp, Dp), jnp.float32),
        jax.ShapeDtypeStruct((Hn * Bp, Dp), jnp.float32),
        jax.ShapeDtypeStruct((Hn * Bp, Dp), jnp.float32),
        jax.ShapeDtypeStruct((Bp, Dp), jnp.float32),
        jax.ShapeDtypeStruct((Bp, Dp), jnp.float32),
    )
    scratch = [
        pltpu.VMEM((BPW,), jnp.int32),
        pltpu.VMEM((BPW,), jnp.int32),
        pltpu.VMEM((RPW,), jnp.int32),
        pltpu.VMEM((BPW, Dm), jnp.float32),
        pltpu.VMEM((NBUF, CHK, Dm), jnp.float32),
        pltpu.SemaphoreType.DMA,
        pltpu.SemaphoreType.DMA,
    ]

    @functools.partial(
        pl.kernel, mesh=mesh, out_type=out_type, scratch_types=scratch,
        compiler_params=pltpu.CompilerParams(use_tc_tiling_on_sc=False))
    def k(nodes_u_h, nodes_v_h, hu_h, so_h, hv_h, u2e_h, v2e_h,
          ehu_o, eso_o, ehv_o, urep_o, vrep_o,
          nu_v, nv_v, ids_v, rep_v, emb_v, gsem, wsem):
        wid = lax.axis_index("s") * 2 + lax.axis_index("c")
        base = wid * BPW
        half = base // Bp
        rowm = base % Bp
        pltpu.sync_copy(nodes_u_h.at[pl.ds(base, BPW)], nu_v)
        pltpu.sync_copy(nodes_v_h.at[pl.ds(base, BPW)], nv_v)
        pltpu.async_copy(u2e_h.at[nu_v], rep_v, gsem).wait()
        pltpu.sync_copy(
            rep_v, urep_o.at[pl.ds(rowm, BPW), pl.ds(half * Dm, Dm)])
        pltpu.async_copy(v2e_h.at[nv_v], rep_v, gsem).wait()
        pltpu.sync_copy(
            rep_v, vrep_o.at[pl.ds(rowm, BPW), pl.ds(half * Dm, Dm)])

        obase = base * Hn

        def dst(out_ref, c):
            s = obase + c * CHK
            h = s // B
            j = s % B
            ch = j // Bp
            m = j % Bp
            return out_ref.at[pl.ds(h * Bp + m, CHK), pl.ds(ch * Dm, Dm)]

        def gather_rows(ids_hbm, emb_tbl, out_ref):
            pltpu.sync_copy(ids_hbm.at[pl.ds(obase, RPW)], ids_v)

            def gstart(c, b):
                pltpu.async_copy(
                    emb_tbl.at[ids_v.at[pl.ds(c * CHK, CHK)]],
                    emb_v.at[b], gsem)

            def gwait(b):
                pltpu.make_async_copy(
                    emb_tbl.at[pl.ds(0, CHK)], emb_v.at[b], gsem).wait()

            def wstart(c, b):
                pltpu.async_copy(emb_v.at[b], dst(out_ref, c), wsem)

            def wwait(b):
                pltpu.make_async_copy(
                    emb_v.at[b], dst(out_ref, 0), wsem).wait()

            def body(c, carry):
                b = lax.rem(c, NBUF)

                @pl.when(c >= NBUF)
                def _():
                    wwait(b)

                gstart(c, b)

                @pl.when(c >= 1)
                def _():
                    bp = lax.rem(c - 1, NBUF)
                    gwait(bp)
                    wstart(c - 1, bp)
                return carry

            lax.fori_loop(0, NCHK, body, 0)
            bl = (NCHK - 1) % NBUF
            gwait(bl)
            wstart(NCHK - 1, bl)
            for _ in range(min(NBUF, NCHK)):
                wwait(0)

        gather_rows(hu_h, v2e_h, ehu_o)
        gather_rows(so_h, u2e_h, eso_o)
        gather_rows(hv_h, u2e_h, ehv_o)

    return k(nodes_u, nodes_v, hu_ids, soc_ids, hv_ids, u2e, v2e)


def _dot(x, w):
    return jax.lax.dot_general(x, w, (((1,), (0,)), ((), ())),
                               preferred_element_type=jnp.float32)


def _relu(x):
    return jnp.maximum(x, 0.0)


def _tc_enc_body(ehu_r, eso_r, ehv_r, hur_r, hvr_r, urep_r, vrep_r,
                 ua_w1e_r, ua_m25_r, ua_b1_r, ua_w2_r, ua_b2_r,
                 ua_a1o_r, ua_a1r_r, ua_ab1_r, ua_a2_r, ua_ab2_r, ua_a3_r,
                 ue_wr_r, ue_wa_r, ue_b_r,
                 sa_a1o_r, sa_a1r_r, sa_ab1_r, sa_a2_r, sa_ab2_r, sa_a3_r,
                 se_wr_r, se_wa_r, se_b_r,
                 va_w1e_r, va_m25_r, va_b1_r, va_w2_r, va_b2_r,
                 va_a1o_r, va_a1r_r, va_ab1_r, va_a2_r, va_ab2_r, va_a3_r,
                 ve_wr_r, ve_wa_r, ve_b_r,
                 eu_o, ev_o):
    def flat(x3):
        return x3.reshape(Hn * BBp, Dp)

    def onehot(ids3):
        i = jax.lax.broadcasted_iota(jnp.int32, (1, 1, NR * NR), 2)
        return (ids3[:, :, None] == i).astype(jnp.float32).reshape(
            Hn * BBp, NR * NR)

    def att_pool(o_flat, rep, a1o, a1r, ab1, a2, ab2, a3):
        a_r = _dot(rep, a1r[...]) + ab1[...]
        a = _dot(o_flat, a1o[...]).reshape(Hn, BBp, Dp)
        a = _relu(a + a_r[None])
        a = _relu(_dot(flat(a), a2[...]) + ab2[...])
        lg = _dot(a, a3[...]).reshape(Hn, BBp, 2)
        m = jnp.max(lg, axis=0, keepdims=True)
        e = jnp.exp(lg - m)
        att = e / jnp.sum(e, axis=0, keepdims=True)
        return jnp.concatenate(
            [jnp.broadcast_to(att[:, :, 0:1], (Hn, BBp, Dm)),
             jnp.broadcast_to(att[:, :, 1:2], (Hn, BBp, Dm))], axis=-1)

    def uv_encode(e3, ids, rep, w1e, m25, b1, w2, b2,
                  a1o, a1r, ab1, a2, ab2, a3, ewr, ewa, eb):
        x = _relu(_dot(flat(e3), w1e[...]) + _dot(onehot(ids), m25[...])
                  + b1[...])
        o = _relu(_dot(x, w2[...]) + b2[...])
        att = att_pool(o, rep, a1o, a1r, ab1, a2, ab2, a3)
        agg = jnp.sum(att * o.reshape(Hn, BBp, Dp), axis=0)
        return _relu(_dot(rep, ewr[...]) + _dot(agg, ewa[...]) + eb[...])

    urep = urep_r[...]
    vrep = vrep_r[...]
    u_hist = uv_encode(ehu_r[...], hur_r[...], urep,
                       ua_w1e_r, ua_m25_r, ua_b1_r, ua_w2_r, ua_b2_r,
                       ua_a1o_r, ua_a1r_r, ua_ab1_r, ua_a2_r, ua_ab2_r,
                       ua_a3_r, ue_wr_r, ue_wa_r, ue_b_r)
    neigh = eso_r[...]
    aw = att_pool(flat(neigh), urep,
                  sa_a1o_r, sa_a1r_r, sa_ab1_r, sa_a2_r, sa_ab2_r, sa_a3_r)
    soc_agg = jnp.sum(aw * neigh, axis=0)
    eu_o[...] = _relu(_dot(u_hist, se_wr_r[...]) + _dot(soc_agg, se_wa_r[...])
                      + se_b_r[...])
    ev_o[...] = uv_encode(ehv_r[...], hvr_r[...], vrep,
                          va_w1e_r, va_m25_r, va_b1_r, va_w2_r, va_b2_r,
                          va_a1o_r, va_a1r_r, va_ab1_r, va_a2_r, va_ab2_r,
                          va_a3_r, ve_wr_r, ve_wa_r, ve_b_r)


def _tc_encoders(ehu, eso, ehv, hur, hvr, urep, vrep, wts):
    bh3 = pl.BlockSpec((Hn, BBp, Dp), lambda i: (0, i, 0))
    bh2 = pl.BlockSpec((Hn, BBp), lambda i: (0, i))
    bb2 = pl.BlockSpec((BBp, Dp), lambda i: (i, 0))

    def full2(a):
        return pl.BlockSpec(a.shape, lambda i: (0, 0))

    in_specs = ([bh3] * 3 + [bh2] * 2 + [bb2] * 2 + [full2(w) for w in wts])
    out_specs = (bb2, bb2)
    out_shape = (jax.ShapeDtypeStruct((Bp, Dp), jnp.float32),
                 jax.ShapeDtypeStruct((Bp, Dp), jnp.float32))
    return pl.pallas_call(
        _tc_enc_body,
        grid=(NB,),
        in_specs=in_specs,
        out_specs=out_specs,
        out_shape=out_shape,
    )(ehu, eso, ehv, hur, hvr, urep, vrep, *wts)


def _tc_head_body(eu_r, ev_r,
                  ur1_w_r, ur1_b_r, ur2_w_r, ur2_b_r,
                  vr1_w_r, vr1_b_r, vr2_w_r, vr2_b_r,
                  uv1u_w_r, uv1v_w_r, uv1_b_r, uv2_w_r, uv2_b_r,
                  uv3_w_r, uv3_b_r,
                  bn1_g_r, bn1_b_r, bn2_g_r, bn2_b_r,
                  bn3_g_r, bn3_b_r, bn4_g_r, bn4_b_r,
                  out_o):
    def bn(x, g, b, hw):
        s = jnp.mean(x, axis=0, keepdims=True)
        m = (s[:, :hw] + s[:, hw:]) * 0.5
        m = jnp.concatenate([m, m], axis=1)
        d = x - m
        v = jnp.mean(d * d, axis=0, keepdims=True)
        v = (v[:, :hw] + v[:, hw:]) * 0.5
        v = jnp.concatenate([v, v], axis=1)
        return g[...] * d * jax.lax.rsqrt(v + 1e-5) + b[...]

    x_u = jnp.maximum(bn(_dot(eu_r[...], ur1_w_r[...]) + ur1_b_r[...],
                         bn1_g_r, bn1_b_r, Dm), 0.0)
    x_u = _dot(x_u, ur2_w_r[...]) + ur2_b_r[...]
    x_v = jnp.maximum(bn(_dot(ev_r[...], vr1_w_r[...]) + vr1_b_r[...],
                         bn2_g_r, bn2_b_r, Dm), 0.0)
    x_v = _dot(x_v, vr2_w_r[...]) + vr2_b_r[...]
    x = _dot(x_u, uv1u_w_r[...]) + _dot(x_v, uv1v_w_r[...]) + uv1_b_r[...]
    x = jnp.maximum(bn(x, bn3_g_r, bn3_b_r, Dm), 0.0)
    x = jnp.maximum(bn(_dot(x, uv2_w_r[...]) + uv2_b_r[...],
                       bn4_g_r, bn4_b_r, 16), 0.0)
    out_o[...] = _dot(x, uv3_w_r[...]) + uv3_b_r[0, 0]


def _tc_head(eu, ev, wts):
    in_specs = [pl.BlockSpec(a.shape, lambda: tuple(0 for _ in a.shape))
                for a in (eu, ev, *wts)]
    return pl.pallas_call(
        _tc_head_body,
        in_specs=in_specs,
        out_specs=pl.BlockSpec((Bp, 2), lambda: (0, 0)),
        out_shape=jax.ShapeDtypeStruct((Bp, 2), jnp.float32),
    )(eu, ev, *wts)


def _bdiag(w):
    zk = jnp.zeros_like(w)
    return jnp.concatenate(
        [jnp.concatenate([w, zk], axis=1),
         jnp.concatenate([zk, w], axis=1)], axis=0)


def _brow(v):
    r = v.reshape(1, -1)
    return jnp.concatenate([r, r], axis=1)


def kernel(nodes_u, nodes_v, hist_u_v, hist_u_r, hist_v_u, hist_v_r,
           social_adj, params):
    p = params
    hu_ids = jnp.take(hist_u_v, nodes_u, axis=0).T.reshape(-1)
    soc_ids = jnp.take(social_adj, nodes_u, axis=0).T.reshape(-1)
    hv_ids = jnp.take(hist_v_u, nodes_v, axis=0).T.reshape(-1)
    hur = jnp.take(hist_u_r, nodes_u, axis=0).T
    hvr = jnp.take(hist_v_r, nodes_v, axis=0).T
    hur_p = hur[:, :Bp] * NR + hur[:, Bp:]
    hvr_p = hvr[:, :Bp] * NR + hvr[:, Bp:]
    ehu, eso, ehv, urep, vrep = _sc_gather(
        nodes_u, nodes_v, hu_ids, soc_ids, hv_ids, p['u2e'], p['v2e'])
    ehu = ehu.reshape(Hn, Bp, Dp)
    eso = eso.reshape(Hn, Bp, Dp)
    ehv = ehv.reshape(Hn, Bp, Dp)

    ua, sa, va = p['ua'], p['sa'], p['va']

    def att_wts(a):
        a3 = a['att3_w'][:, 0]
        z = jnp.zeros_like(a3)
        a3p = jnp.stack([jnp.concatenate([a3, z]),
                         jnp.concatenate([z, a3])], axis=1)
        return [_bdiag(a['att1_w'][:Dm]), _bdiag(a['att1_w'][Dm:]),
                _brow(a['att1_b']), _bdiag(a['att2_w']),
                _brow(a['att2_b']), a3p]

    def agg_wts(a, ew, eb):
        m1 = p['r2e'] @ a['w_r1_w'][Dm:]
        m25 = jnp.concatenate(
            [jnp.repeat(m1, NR, axis=0), jnp.tile(m1, (NR, 1))], axis=1)
        return ([_bdiag(a['w_r1_w'][:Dm]), m25,
                 _brow(a['w_r1_b']), _bdiag(a['w_r2_w']), _brow(a['w_r2_b'])]
                + att_wts(a)
                + [_bdiag(ew[:Dm]), _bdiag(ew[Dm:]), _brow(eb)])

    enc_wts = (agg_wts(ua, p['ue_w'], p['ue_b'])
               + att_wts(sa) + [_bdiag(p['se_w'][:Dm]), _bdiag(p['se_w'][Dm:]),
                                _brow(p['se_b'])]
               + agg_wts(va, p['ve_w'], p['ve_b']))
    eu, ev = _tc_encoders(ehu, eso, ehv, hur_p, hvr_p, urep, vrep, enc_wts)

    w3 = p['w_uv3_w']
    z3 = jnp.zeros_like(w3)
    uv3_p = jnp.concatenate(
        [jnp.concatenate([w3, z3], axis=1),
         jnp.concatenate([z3, w3], axis=1)], axis=0)
    head_wts = [
        _bdiag(p['w_ur1_w']), _brow(p['w_ur1_b']),
        _bdiag(p['w_ur2_w']), _brow(p['w_ur2_b']),
        _bdiag(p['w_vr1_w']), _brow(p['w_vr1_b']),
        _bdiag(p['w_vr2_w']), _brow(p['w_vr2_b']),
        _bdiag(p['w_uv1_w'][:Dm]), _bdiag(p['w_uv1_w'][Dm:]),
        _brow(p['w_uv1_b']),
        _bdiag(p['w_uv2_w']), _brow(p['w_uv2_b']),
        uv3_p, p['w_uv3_b'].reshape(1, 1),
        _brow(p['bn1_g']), _brow(p['bn1_b']),
        _brow(p['bn2_g']), _brow(p['bn2_b']),
        _brow(p['bn3_g']), _brow(p['bn3_b']),
        _brow(p['bn4_g']), _brow(p['bn4_b']),
    ]
    out = _tc_head(eu, ev, head_wts)
    return jnp.concatenate([out[:, 0], out[:, 1]])


# speedup vs baseline: 8.1038x; 1.0012x over previous
import functools

import jax
import jax.numpy as jnp
from jax import lax
from jax.experimental import pallas as pl
from jax.experimental.pallas import tpu as pltpu
from jax.experimental.pallas import tpu_sc as plsc

B = 4096
Bp = B // 2
Hn = 50
Dm = 64
Dp = 2 * Dm
NR = 5

NW = 32
BPW = B // NW
RPW = BPW * Hn
CHK = 128
NCHK = RPW // CHK
NBUF = 4

BBp = 128
NB = Bp // BBp


def _sc_gather(nodes_u, nodes_v, hu_ids, soc_ids, hv_ids, u2e, v2e):
    mesh = plsc.VectorSubcoreMesh(core_axis_name="c", subcore_axis_name="s")
    out_type = (
        jax.ShapeDtypeStruct((Hn, Bp, Dp), jnp.float32),
        jax.ShapeDtypeStruct((Hn, Bp, Dp), jnp.float32),
        jax.ShapeDtypeStruct((Hn, Bp, Dp), jnp.float32),
        jax.ShapeDtypeStruct((Bp, Dp), jnp.float32),
        jax.ShapeDtypeStruct((Bp, Dp), jnp.float32),
    )
    scratch = [
        pltpu.VMEM((BPW,), jnp.int32),
        pltpu.VMEM((BPW,), jnp.int32),
        pltpu.VMEM((RPW,), jnp.int32),
        pltpu.VMEM((BPW, Dm), jnp.float32),
        pltpu.VMEM((NBUF, CHK, Dm), jnp.float32),
        pltpu.SemaphoreType.DMA,
        pltpu.SemaphoreType.DMA,
    ]

    @functools.partial(
        pl.kernel, mesh=mesh, out_type=out_type, scratch_types=scratch,
        compiler_params=pltpu.CompilerParams(use_tc_tiling_on_sc=False))
    def k(nodes_u_h, nodes_v_h, hu_h, so_h, hv_h, u2e_h, v2e_h,
          ehu_o, eso_o, ehv_o, urep_o, vrep_o,
          nu_v, nv_v, ids_v, rep_v, emb_v, gsem, wsem):
        wid = lax.axis_index("s") * 2 + lax.axis_index("c")
        base = wid * BPW
        half = base // Bp
        rowm = base % Bp
        pltpu.sync_copy(nodes_u_h.at[pl.ds(base, BPW)], nu_v)
        pltpu.sync_copy(nodes_v_h.at[pl.ds(base, BPW)], nv_v)
        pltpu.async_copy(u2e_h.at[nu_v], rep_v, gsem).wait()
        pltpu.sync_copy(
            rep_v, urep_o.at[pl.ds(rowm, BPW), pl.ds(half * Dm, Dm)])
        pltpu.async_copy(v2e_h.at[nv_v], rep_v, gsem).wait()
        pltpu.sync_copy(
            rep_v, vrep_o.at[pl.ds(rowm, BPW), pl.ds(half * Dm, Dm)])

        obase = base * Hn

        def dst(out_ref, c):
            s = obase + c * CHK
            h = s // B
            j = s % B
            ch = j // Bp
            m = j % Bp
            return out_ref.at[h, pl.ds(m, CHK), pl.ds(ch * Dm, Dm)]

        def gather_rows(ids_hbm, emb_tbl, out_ref):
            pltpu.sync_copy(ids_hbm.at[pl.ds(obase, RPW)], ids_v)

            def gstart(c, b):
                pltpu.async_copy(
                    emb_tbl.at[ids_v.at[pl.ds(c * CHK, CHK)]],
                    emb_v.at[b], gsem)

            def gwait(b):
                pltpu.make_async_copy(
                    emb_tbl.at[pl.ds(0, CHK)], emb_v.at[b], gsem).wait()

            def wstart(c, b):
                pltpu.async_copy(emb_v.at[b], dst(out_ref, c), wsem)

            def wwait(b):
                pltpu.make_async_copy(
                    emb_v.at[b], out_ref.at[0, pl.ds(0, CHK), pl.ds(0, Dm)],
                    wsem).wait()

            def body(c, carry):
                b = lax.rem(c, NBUF)

                @pl.when(c >= NBUF)
                def _():
                    wwait(b)

                gstart(c, b)

                @pl.when(c >= 1)
                def _():
                    bp = lax.rem(c - 1, NBUF)
                    gwait(bp)
                    wstart(c - 1, bp)
                return carry

            lax.fori_loop(0, NCHK, body, 0)
            bl = (NCHK - 1) % NBUF
            gwait(bl)
            wstart(NCHK - 1, bl)
            for _ in range(min(NBUF, NCHK)):
                wwait(0)

        gather_rows(hu_h, v2e_h, ehu_o)
        gather_rows(so_h, u2e_h, eso_o)
        gather_rows(hv_h, u2e_h, ehv_o)

    return k(nodes_u, nodes_v, hu_ids, soc_ids, hv_ids, u2e, v2e)


def _dot(x, w):
    return jax.lax.dot_general(x, w, (((1,), (0,)), ((), ())),
                               preferred_element_type=jnp.float32)


def _relu(x):
    return jnp.maximum(x, 0.0)


def _tc_enc_body(ehu_r, eso_r, ehv_r, hur_r, hvr_r, urep_r, vrep_r,
                 ua_w1e_r, ua_m25_r, ua_b1_r, ua_w2_r, ua_b2_r,
                 ua_a1o_r, ua_a1r_r, ua_ab1_r, ua_a2_r, ua_ab2_r, ua_a3_r,
                 ue_wr_r, ue_wa_r, ue_b_r,
                 sa_a1o_r, sa_a1r_r, sa_ab1_r, sa_a2_r, sa_ab2_r, sa_a3_r,
                 se_wr_r, se_wa_r, se_b_r,
                 va_w1e_r, va_m25_r, va_b1_r, va_w2_r, va_b2_r,
                 va_a1o_r, va_a1r_r, va_ab1_r, va_a2_r, va_ab2_r, va_a3_r,
                 ve_wr_r, ve_wa_r, ve_b_r,
                 eu_o, ev_o):
    def flat(x3):
        return x3.reshape(Hn * BBp, Dp)

    def onehot(ids3):
        i = jax.lax.broadcasted_iota(jnp.int32, (1, 1, NR * NR), 2)
        return (ids3[:, :, None] == i).astype(jnp.float32).reshape(
            Hn * BBp, NR * NR)

    def att_pool(o_flat, rep, a1o, a1r, ab1, a2, ab2, a3):
        a_r = _dot(rep, a1r[...]) + ab1[...]
        a = _dot(o_flat, a1o[...]).reshape(Hn, BBp, Dp)
        a = _relu(a + a_r[None])
        a = _relu(_dot(flat(a), a2[...]) + ab2[...])
        lg = _dot(a, a3[...]).reshape(Hn, BBp, 2)
        m = jnp.max(lg, axis=0, keepdims=True)
        e = jnp.exp(lg - m)
        att = e / jnp.sum(e, axis=0, keepdims=True)
        return jnp.concatenate(
            [jnp.broadcast_to(att[:, :, 0:1], (Hn, BBp, Dm)),
             jnp.broadcast_to(att[:, :, 1:2], (Hn, BBp, Dm))], axis=-1)

    def uv_encode(e3, ids, rep, w1e, m25, b1, w2, b2,
                  a1o, a1r, ab1, a2, ab2, a3, ewr, ewa, eb):
        x = _relu(_dot(flat(e3), w1e[...]) + _dot(onehot(ids), m25[...])
                  + b1[...])
        o = _relu(_dot(x, w2[...]) + b2[...])
        att = att_pool(o, rep, a1o, a1r, ab1, a2, ab2, a3)
        agg = jnp.sum(att * o.reshape(Hn, BBp, Dp), axis=0)
        return _relu(_dot(rep, ewr[...]) + _dot(agg, ewa[...]) + eb[...])

    urep = urep_r[...]
    vrep = vrep_r[...]
    u_hist = uv_encode(ehu_r[...], hur_r[...], urep,
                       ua_w1e_r, ua_m25_r, ua_b1_r, ua_w2_r, ua_b2_r,
                       ua_a1o_r, ua_a1r_r, ua_ab1_r, ua_a2_r, ua_ab2_r,
                       ua_a3_r, ue_wr_r, ue_wa_r, ue_b_r)
    neigh = eso_r[...]
    aw = att_pool(flat(neigh), urep,
                  sa_a1o_r, sa_a1r_r, sa_ab1_r, sa_a2_r, sa_ab2_r, sa_a3_r)
    soc_agg = jnp.sum(aw * neigh, axis=0)
    eu_o[...] = _relu(_dot(u_hist, se_wr_r[...]) + _dot(soc_agg, se_wa_r[...])
                      + se_b_r[...])
    ev_o[...] = uv_encode(ehv_r[...], hvr_r[...], vrep,
                          va_w1e_r, va_m25_r, va_b1_r, va_w2_r, va_b2_r,
                          va_a1o_r, va_a1r_r, va_ab1_r, va_a2_r, va_ab2_r,
                          va_a3_r, ve_wr_r, ve_wa_r, ve_b_r)


def _tc_encoders(ehu, eso, ehv, hur, hvr, urep, vrep, wts):
    bh3 = pl.BlockSpec((Hn, BBp, Dp), lambda i: (0, i, 0))
    bh2 = pl.BlockSpec((Hn, BBp), lambda i: (0, i))
    bb2 = pl.BlockSpec((BBp, Dp), lambda i: (i, 0))

    def full2(a):
        return pl.BlockSpec(a.shape, lambda i: (0, 0))

    in_specs = ([bh3] * 3 + [bh2] * 2 + [bb2] * 2 + [full2(w) for w in wts])
    out_specs = (bb2, bb2)
    out_shape = (jax.ShapeDtypeStruct((Bp, Dp), jnp.float32),
                 jax.ShapeDtypeStruct((Bp, Dp), jnp.float32))
    return pl.pallas_call(
        _tc_enc_body,
        grid=(NB,),
        in_specs=in_specs,
        out_specs=out_specs,
        out_shape=out_shape,
    )(ehu, eso, ehv, hur, hvr, urep, vrep, *wts)


def _tc_head_body(eu_r, ev_r,
                  ur1_w_r, ur1_b_r, ur2_w_r, ur2_b_r,
                  vr1_w_r, vr1_b_r, vr2_w_r, vr2_b_r,
                  uv1u_w_r, uv1v_w_r, uv1_b_r, uv2_w_r, uv2_b_r,
                  uv3_w_r, uv3_b_r,
                  bn1_g_r, bn1_b_r, bn2_g_r, bn2_b_r,
                  bn3_g_r, bn3_b_r, bn4_g_r, bn4_b_r,
                  out_o):
    def bn(x, g, b, hw):
        s = jnp.mean(x, axis=0, keepdims=True)
        m = (s[:, :hw] + s[:, hw:]) * 0.5
        m = jnp.concatenate([m, m], axis=1)
        d = x - m
        v = jnp.mean(d * d, axis=0, keepdims=True)
        v = (v[:, :hw] + v[:, hw:]) * 0.5
        v = jnp.concatenate([v, v], axis=1)
        return g[...] * d * jax.lax.rsqrt(v + 1e-5) + b[...]

    x_u = jnp.maximum(bn(_dot(eu_r[...], ur1_w_r[...]) + ur1_b_r[...],
                         bn1_g_r, bn1_b_r, Dm), 0.0)
    x_u = _dot(x_u, ur2_w_r[...]) + ur2_b_r[...]
    x_v = jnp.maximum(bn(_dot(ev_r[...], vr1_w_r[...]) + vr1_b_r[...],
                         bn2_g_r, bn2_b_r, Dm), 0.0)
    x_v = _dot(x_v, vr2_w_r[...]) + vr2_b_r[...]
    x = _dot(x_u, uv1u_w_r[...]) + _dot(x_v, uv1v_w_r[...]) + uv1_b_r[...]
    x = jnp.maximum(bn(x, bn3_g_r, bn3_b_r, Dm), 0.0)
    x = jnp.maximum(bn(_dot(x, uv2_w_r[...]) + uv2_b_r[...],
                       bn4_g_r, bn4_b_r, 16), 0.0)
    out_o[...] = _dot(x, uv3_w_r[...]) + uv3_b_r[0, 0]


def _tc_head(eu, ev, wts):
    in_specs = [pl.BlockSpec(a.shape, lambda: tuple(0 for _ in a.shape))
                for a in (eu, ev, *wts)]
    return pl.pallas_call(
        _tc_head_body,
        in_specs=in_specs,
        out_specs=pl.BlockSpec((Bp, 2), lambda: (0, 0)),
        out_shape=jax.ShapeDtypeStruct((Bp, 2), jnp.float32),
    )(eu, ev, *wts)


def _bdiag(w):
    zk = jnp.zeros_like(w)
    return jnp.concatenate(
        [jnp.concatenate([w, zk], axis=1),
         jnp.concatenate([zk, w], axis=1)], axis=0)


def _brow(v):
    r = v.reshape(1, -1)
    return jnp.concatenate([r, r], axis=1)


def kernel(nodes_u, nodes_v, hist_u_v, hist_u_r, hist_v_u, hist_v_r,
           social_adj, params):
    p = params
    hu_ids = jnp.take(hist_u_v, nodes_u, axis=0).T.reshape(-1)
    soc_ids = jnp.take(social_adj, nodes_u, axis=0).T.reshape(-1)
    hv_ids = jnp.take(hist_v_u, nodes_v, axis=0).T.reshape(-1)
    hur = jnp.take(hist_u_r, nodes_u, axis=0).T
    hvr = jnp.take(hist_v_r, nodes_v, axis=0).T
    hur_p = hur[:, :Bp] * NR + hur[:, Bp:]
    hvr_p = hvr[:, :Bp] * NR + hvr[:, Bp:]
    ehu, eso, ehv, urep, vrep = _sc_gather(
        nodes_u, nodes_v, hu_ids, soc_ids, hv_ids, p['u2e'], p['v2e'])

    ua, sa, va = p['ua'], p['sa'], p['va']

    def att_wts(a):
        a3 = a['att3_w'][:, 0]
        z = jnp.zeros_like(a3)
        a3p = jnp.stack([jnp.concatenate([a3, z]),
                         jnp.concatenate([z, a3])], axis=1)
        return [_bdiag(a['att1_w'][:Dm]), _bdiag(a['att1_w'][Dm:]),
                _brow(a['att1_b']), _bdiag(a['att2_w']),
                _brow(a['att2_b']), a3p]

    def agg_wts(a, ew, eb):
        m1 = p['r2e'] @ a['w_r1_w'][Dm:]
        m25 = jnp.concatenate(
            [jnp.repeat(m1, NR, axis=0), jnp.tile(m1, (NR, 1))], axis=1)
        return ([_bdiag(a['w_r1_w'][:Dm]), m25,
                 _brow(a['w_r1_b']), _bdiag(a['w_r2_w']), _brow(a['w_r2_b'])]
                + att_wts(a)
                + [_bdiag(ew[:Dm]), _bdiag(ew[Dm:]), _brow(eb)])

    enc_wts = (agg_wts(ua, p['ue_w'], p['ue_b'])
               + att_wts(sa) + [_bdiag(p['se_w'][:Dm]), _bdiag(p['se_w'][Dm:]),
                                _brow(p['se_b'])]
               + agg_wts(va, p['ve_w'], p['ve_b']))
    eu, ev = _tc_encoders(ehu, eso, ehv, hur_p, hvr_p, urep, vrep, enc_wts)

    w3 = p['w_uv3_w']
    z3 = jnp.zeros_like(w3)
    uv3_p = jnp.concatenate(
        [jnp.concatenate([w3, z3], axis=1),
         jnp.concatenate([z3, w3], axis=1)], axis=0)
    head_wts = [
        _bdiag(p['w_ur1_w']), _brow(p['w_ur1_b']),
        _bdiag(p['w_ur2_w']), _brow(p['w_ur2_b']),
        _bdiag(p['w_vr1_w']), _brow(p['w_vr1_b']),
        _bdiag(p['w_vr2_w']), _brow(p['w_vr2_b']),
        _bdiag(p['w_uv1_w'][:Dm]), _bdiag(p['w_uv1_w'][Dm:]),
        _brow(p['w_uv1_b']),
        _bdiag(p['w_uv2_w']), _brow(p['w_uv2_b']),
        uv3_p, p['w_uv3_b'].reshape(1, 1),
        _brow(p['bn1_g']), _brow(p['bn1_b']),
        _brow(p['bn2_g']), _brow(p['bn2_b']),
        _brow(p['bn3_g']), _brow(p['bn3_b']),
        _brow(p['bn4_g']), _brow(p['bn4_b']),
    ]
    out = _tc_head(eu, ev, head_wts)
    return jnp.concatenate([out[:, 0], out[:, 1]])


# speedup vs baseline: 8.4149x; 1.0384x over previous
import functools

import jax
import jax.numpy as jnp
from jax import lax
from jax.experimental import pallas as pl
from jax.experimental.pallas import tpu as pltpu
from jax.experimental.pallas import tpu_sc as plsc

B = 4096
Bp = B // 2
Hn = 50
Dm = 64
Dp = 2 * Dm
NR = 5

NW = 32
CHK = 128
NBUF = 4

BBp = 128


def _sc_gather(nodes_u, nodes_v, hu_ids, soc_ids, hv_ids, u2e, v2e):
    b = nodes_u.shape[0]
    bp = b // 2
    bpw = b // NW
    rpw = bpw * Hn
    nchk = rpw // CHK
    mesh = plsc.VectorSubcoreMesh(core_axis_name="c", subcore_axis_name="s")
    out_type = (
        jax.ShapeDtypeStruct((Hn, bp, Dp), jnp.float32),
        jax.ShapeDtypeStruct((Hn, bp, Dp), jnp.float32),
        jax.ShapeDtypeStruct((Hn, bp, Dp), jnp.float32),
        jax.ShapeDtypeStruct((bp, Dp), jnp.float32),
        jax.ShapeDtypeStruct((bp, Dp), jnp.float32),
    )
    scratch = [
        pltpu.VMEM((bpw,), jnp.int32),
        pltpu.VMEM((bpw,), jnp.int32),
        pltpu.VMEM((rpw,), jnp.int32),
        pltpu.VMEM((bpw, Dm), jnp.float32),
        pltpu.VMEM((NBUF, CHK, Dm), jnp.float32),
        pltpu.SemaphoreType.DMA,
        pltpu.SemaphoreType.DMA,
    ]

    @functools.partial(
        pl.kernel, mesh=mesh, out_type=out_type, scratch_types=scratch,
        compiler_params=pltpu.CompilerParams(use_tc_tiling_on_sc=False))
    def k(nodes_u_h, nodes_v_h, hu_h, so_h, hv_h, u2e_h, v2e_h,
          ehu_o, eso_o, ehv_o, urep_o, vrep_o,
          nu_v, nv_v, ids_v, rep_v, emb_v, gsem, wsem):
        wid = lax.axis_index("s") * 2 + lax.axis_index("c")
        base = wid * bpw
        half = base // bp
        rowm = base % bp
        pltpu.sync_copy(nodes_u_h.at[pl.ds(base, bpw)], nu_v)
        pltpu.sync_copy(nodes_v_h.at[pl.ds(base, bpw)], nv_v)
        pltpu.async_copy(u2e_h.at[nu_v], rep_v, gsem).wait()
        pltpu.sync_copy(
            rep_v, urep_o.at[pl.ds(rowm, bpw), pl.ds(half * Dm, Dm)])
        pltpu.async_copy(v2e_h.at[nv_v], rep_v, gsem).wait()
        pltpu.sync_copy(
            rep_v, vrep_o.at[pl.ds(rowm, bpw), pl.ds(half * Dm, Dm)])

        obase = base * Hn

        def dst(out_ref, c):
            s = obase + c * CHK
            h = s // b
            j = s % b
            ch = j // bp
            m = j % bp
            return out_ref.at[h, pl.ds(m, CHK), pl.ds(ch * Dm, Dm)]

        def gather_rows(ids_hbm, emb_tbl, out_ref):
            pltpu.sync_copy(ids_hbm.at[pl.ds(obase, rpw)], ids_v)

            def gstart(c, b):
                pltpu.async_copy(
                    emb_tbl.at[ids_v.at[pl.ds(c * CHK, CHK)]],
                    emb_v.at[b], gsem)

            def gwait(b):
                pltpu.make_async_copy(
                    emb_tbl.at[pl.ds(0, CHK)], emb_v.at[b], gsem).wait()

            def wstart(c, b):
                pltpu.async_copy(emb_v.at[b], dst(out_ref, c), wsem)

            def wwait(b):
                pltpu.make_async_copy(
                    emb_v.at[b], out_ref.at[0, pl.ds(0, CHK), pl.ds(0, Dm)],
                    wsem).wait()

            def body(c, carry):
                b = lax.rem(c, NBUF)

                @pl.when(c >= NBUF)
                def _():
                    wwait(b)

                gstart(c, b)

                @pl.when(c >= 1)
                def _():
                    bp = lax.rem(c - 1, NBUF)
                    gwait(bp)
                    wstart(c - 1, bp)
                return carry

            lax.fori_loop(0, nchk, body, 0)
            bl = (nchk - 1) % NBUF
            gwait(bl)
            wstart(nchk - 1, bl)
            for _ in range(min(NBUF, nchk)):
                wwait(0)

        gather_rows(hu_h, v2e_h, ehu_o)
        gather_rows(so_h, u2e_h, eso_o)
        gather_rows(hv_h, u2e_h, ehv_o)

    return k(nodes_u, nodes_v, hu_ids, soc_ids, hv_ids, u2e, v2e)


def _dot(x, w):
    return jax.lax.dot_general(x, w, (((1,), (0,)), ((), ())),
                               preferred_element_type=jnp.float32)


def _relu(x):
    return jnp.maximum(x, 0.0)


def _tc_enc_body(ehu_r, eso_r, ehv_r, hur_r, hvr_r, urep_r, vrep_r,
                 ua_w1e_r, ua_m25_r, ua_b1_r, ua_w2_r, ua_b2_r,
                 ua_a1o_r, ua_a1r_r, ua_ab1_r, ua_a2_r, ua_ab2_r, ua_a3_r,
                 ue_wr_r, ue_wa_r, ue_b_r,
                 sa_a1o_r, sa_a1r_r, sa_ab1_r, sa_a2_r, sa_ab2_r, sa_a3_r,
                 se_wr_r, se_wa_r, se_b_r,
                 va_w1e_r, va_m25_r, va_b1_r, va_w2_r, va_b2_r,
                 va_a1o_r, va_a1r_r, va_ab1_r, va_a2_r, va_ab2_r, va_a3_r,
                 ve_wr_r, ve_wa_r, ve_b_r,
                 eu_o, ev_o):
    def flat(x3):
        return x3.reshape(Hn * BBp, Dp)

    def onehot(ids3):
        i = jax.lax.broadcasted_iota(jnp.int32, (1, 1, NR * NR), 2)
        return (ids3[:, :, None] == i).astype(jnp.float32).reshape(
            Hn * BBp, NR * NR)

    def att_pool(o_flat, rep, a1o, a1r, ab1, a2, ab2, a3):
        a_r = _dot(rep, a1r[...]) + ab1[...]
        a = _dot(o_flat, a1o[...]).reshape(Hn, BBp, Dp)
        a = _relu(a + a_r[None])
        a = _relu(_dot(flat(a), a2[...]) + ab2[...])
        lg = _dot(a, a3[...]).reshape(Hn, BBp, 2)
        m = jnp.max(lg, axis=0, keepdims=True)
        e = jnp.exp(lg - m)
        att = e / jnp.sum(e, axis=0, keepdims=True)
        return jnp.concatenate(
            [jnp.broadcast_to(att[:, :, 0:1], (Hn, BBp, Dm)),
             jnp.broadcast_to(att[:, :, 1:2], (Hn, BBp, Dm))], axis=-1)

    def uv_encode(e3, ids, rep, w1e, m25, b1, w2, b2,
                  a1o, a1r, ab1, a2, ab2, a3, ewr, ewa, eb):
        x = _relu(_dot(flat(e3), w1e[...]) + _dot(onehot(ids), m25[...])
                  + b1[...])
        o = _relu(_dot(x, w2[...]) + b2[...])
        att = att_pool(o, rep, a1o, a1r, ab1, a2, ab2, a3)
        agg = jnp.sum(att * o.reshape(Hn, BBp, Dp), axis=0)
        return _relu(_dot(rep, ewr[...]) + _dot(agg, ewa[...]) + eb[...])

    urep = urep_r[...]
    vrep = vrep_r[...]
    u_hist = uv_encode(ehu_r[...], hur_r[...], urep,
                       ua_w1e_r, ua_m25_r, ua_b1_r, ua_w2_r, ua_b2_r,
                       ua_a1o_r, ua_a1r_r, ua_ab1_r, ua_a2_r, ua_ab2_r,
                       ua_a3_r, ue_wr_r, ue_wa_r, ue_b_r)
    neigh = eso_r[...]
    aw = att_pool(flat(neigh), urep,
                  sa_a1o_r, sa_a1r_r, sa_ab1_r, sa_a2_r, sa_ab2_r, sa_a3_r)
    soc_agg = jnp.sum(aw * neigh, axis=0)
    eu_o[...] = _relu(_dot(u_hist, se_wr_r[...]) + _dot(soc_agg, se_wa_r[...])
                      + se_b_r[...])
    ev_o[...] = uv_encode(ehv_r[...], hvr_r[...], vrep,
                          va_w1e_r, va_m25_r, va_b1_r, va_w2_r, va_b2_r,
                          va_a1o_r, va_a1r_r, va_ab1_r, va_a2_r, va_ab2_r,
                          va_a3_r, ve_wr_r, ve_wa_r, ve_b_r)


def _tc_encoders(ehu, eso, ehv, hur, hvr, urep, vrep, wts):
    bh3 = pl.BlockSpec((Hn, BBp, Dp), lambda i: (0, i, 0))
    bh2 = pl.BlockSpec((Hn, BBp), lambda i: (0, i))
    bb2 = pl.BlockSpec((BBp, Dp), lambda i: (i, 0))

    def full2(a):
        return pl.BlockSpec(a.shape, lambda i: (0, 0))

    bp = urep.shape[0]
    in_specs = ([bh3] * 3 + [bh2] * 2 + [bb2] * 2 + [full2(w) for w in wts])
    out_specs = (bb2, bb2)
    out_shape = (jax.ShapeDtypeStruct((bp, Dp), jnp.float32),
                 jax.ShapeDtypeStruct((bp, Dp), jnp.float32))
    return pl.pallas_call(
        _tc_enc_body,
        grid=(bp // BBp,),
        in_specs=in_specs,
        out_specs=out_specs,
        out_shape=out_shape,
    )(ehu, eso, ehv, hur, hvr, urep, vrep, *wts)


def _tc_head_body(eu_r, ev_r,
                  ur1_w_r, ur1_b_r, ur2_w_r, ur2_b_r,
                  vr1_w_r, vr1_b_r, vr2_w_r, vr2_b_r,
                  uv1u_w_r, uv1v_w_r, uv1_b_r, uv2_w_r, uv2_b_r,
                  uv3_w_r, uv3_b_r,
                  bn1_g_r, bn1_b_r, bn2_g_r, bn2_b_r,
                  bn3_g_r, bn3_b_r, bn4_g_r, bn4_b_r,
                  out_o):
    def bn(x, g, b, hw):
        s = jnp.mean(x, axis=0, keepdims=True)
        m = (s[:, :hw] + s[:, hw:]) * 0.5
        m = jnp.concatenate([m, m], axis=1)
        d = x - m
        v = jnp.mean(d * d, axis=0, keepdims=True)
        v = (v[:, :hw] + v[:, hw:]) * 0.5
        v = jnp.concatenate([v, v], axis=1)
        return g[...] * d * jax.lax.rsqrt(v + 1e-5) + b[...]

    x_u = jnp.maximum(bn(_dot(eu_r[...], ur1_w_r[...]) + ur1_b_r[...],
                         bn1_g_r, bn1_b_r, Dm), 0.0)
    x_u = _dot(x_u, ur2_w_r[...]) + ur2_b_r[...]
    x_v = jnp.maximum(bn(_dot(ev_r[...], vr1_w_r[...]) + vr1_b_r[...],
                         bn2_g_r, bn2_b_r, Dm), 0.0)
    x_v = _dot(x_v, vr2_w_r[...]) + vr2_b_r[...]
    x = _dot(x_u, uv1u_w_r[...]) + _dot(x_v, uv1v_w_r[...]) + uv1_b_r[...]
    x = jnp.maximum(bn(x, bn3_g_r, bn3_b_r, Dm), 0.0)
    x = jnp.maximum(bn(_dot(x, uv2_w_r[...]) + uv2_b_r[...],
                       bn4_g_r, bn4_b_r, 16), 0.0)
    out_o[...] = _dot(x, uv3_w_r[...]) + uv3_b_r[0, 0]


def _tc_head(eu, ev, wts):
    in_specs = [pl.BlockSpec(a.shape, lambda: tuple(0 for _ in a.shape))
                for a in (eu, ev, *wts)]
    return pl.pallas_call(
        _tc_head_body,
        in_specs=in_specs,
        out_specs=pl.BlockSpec((Bp, 2), lambda: (0, 0)),
        out_shape=jax.ShapeDtypeStruct((Bp, 2), jnp.float32),
    )(eu, ev, *wts)


def _bdiag(w):
    zk = jnp.zeros_like(w)
    return jnp.concatenate(
        [jnp.concatenate([w, zk], axis=1),
         jnp.concatenate([zk, w], axis=1)], axis=0)


def _brow(v):
    r = v.reshape(1, -1)
    return jnp.concatenate([r, r], axis=1)


def kernel(nodes_u, nodes_v, hist_u_v, hist_u_r, hist_v_u, hist_v_r,
           social_adj, params):
    p = params
    ua, sa, va = p['ua'], p['sa'], p['va']

    def att_wts(a):
        a3 = a['att3_w'][:, 0]
        z = jnp.zeros_like(a3)
        a3p = jnp.stack([jnp.concatenate([a3, z]),
                         jnp.concatenate([z, a3])], axis=1)
        return [_bdiag(a['att1_w'][:Dm]), _bdiag(a['att1_w'][Dm:]),
                _brow(a['att1_b']), _bdiag(a['att2_w']),
                _brow(a['att2_b']), a3p]

    def agg_wts(a, ew, eb):
        m1 = p['r2e'] @ a['w_r1_w'][Dm:]
        m25 = jnp.concatenate(
            [jnp.repeat(m1, NR, axis=0), jnp.tile(m1, (NR, 1))], axis=1)
        return ([_bdiag(a['w_r1_w'][:Dm]), m25,
                 _brow(a['w_r1_b']), _bdiag(a['w_r2_w']), _brow(a['w_r2_b'])]
                + att_wts(a)
                + [_bdiag(ew[:Dm]), _bdiag(ew[Dm:]), _brow(eb)])

    enc_wts = (agg_wts(ua, p['ue_w'], p['ue_b'])
               + att_wts(sa) + [_bdiag(p['se_w'][:Dm]), _bdiag(p['se_w'][Dm:]),
                                _brow(p['se_b'])]
               + agg_wts(va, p['ve_w'], p['ve_b']))

    bh = B // 2
    eus, evs = [], []
    for sl in (slice(0, bh), slice(bh, B)):
        nu, nv = nodes_u[sl], nodes_v[sl]
        hu_ids = jnp.take(hist_u_v, nu, axis=0).T.reshape(-1)
        soc_ids = jnp.take(social_adj, nu, axis=0).T.reshape(-1)
        hv_ids = jnp.take(hist_v_u, nv, axis=0).T.reshape(-1)
        hur = jnp.take(hist_u_r, nu, axis=0).T
        hvr = jnp.take(hist_v_r, nv, axis=0).T
        hp = bh // 2
        hur_p = hur[:, :hp] * NR + hur[:, hp:]
        hvr_p = hvr[:, :hp] * NR + hvr[:, hp:]
        ehu, eso, ehv, urep, vrep = _sc_gather(
            nu, nv, hu_ids, soc_ids, hv_ids, p['u2e'], p['v2e'])
        eu, ev = _tc_encoders(ehu, eso, ehv, hur_p, hvr_p, urep, vrep,
                              enc_wts)
        eus.append(eu)
        evs.append(ev)
    eu = jnp.concatenate(eus, axis=0)
    ev = jnp.concatenate(evs, axis=0)

    w3 = p['w_uv3_w']
    z3 = jnp.zeros_like(w3)
    uv3_p = jnp.concatenate(
        [jnp.concatenate([w3, z3], axis=1),
         jnp.concatenate([z3, w3], axis=1)], axis=0)
    head_wts = [
        _bdiag(p['w_ur1_w']), _brow(p['w_ur1_b']),
        _bdiag(p['w_ur2_w']), _brow(p['w_ur2_b']),
        _bdiag(p['w_vr1_w']), _brow(p['w_vr1_b']),
        _bdiag(p['w_vr2_w']), _brow(p['w_vr2_b']),
        _bdiag(p['w_uv1_w'][:Dm]), _bdiag(p['w_uv1_w'][Dm:]),
        _brow(p['w_uv1_b']),
        _bdiag(p['w_uv2_w']), _brow(p['w_uv2_b']),
        uv3_p, p['w_uv3_b'].reshape(1, 1),
        _brow(p['bn1_g']), _brow(p['bn1_b']),
        _brow(p['bn2_g']), _brow(p['bn2_b']),
        _brow(p['bn3_g']), _brow(p['bn3_b']),
        _brow(p['bn4_g']), _brow(p['bn4_b']),
    ]
    out = _tc_head(eu, ev, head_wts)
    q = B // 4
    return jnp.concatenate([out[:q, 0], out[:q, 1], out[q:, 0], out[q:, 1]])
